# baseline (speedup 1.0000x reference)
import jax
import jax.numpy as jnp
from jax.experimental import pallas as pl
from jax.experimental.pallas import tpu as pltpu


def _gcn_body(adj_hbm, feat_ref, w1_ref, b1_ref, w2_ref, b2_ref, out_ref,
              abuf, g_ref, sem):
    nb = out_ref.shape[0]
    s = abuf.shape[-1]

    def copy(k, slot):
        return pltpu.make_async_copy(adj_hbm.at[k], abuf.at[slot], sem.at[slot])

    copy(0, 0).start()
    g_ref[...] = jax.lax.dot_general(
        feat_ref[...], w1_ref[...], (((2,), (0,)), ((), ())),
        preferred_element_type=jnp.float32).astype(jnp.bfloat16)

    for k in range(nb):
        if k + 1 < nb:
            copy(k + 1, (k + 1) % 2).start()
        copy(k, k % 2).wait()
        adj = abuf[k % 2]
        deg = jnp.maximum(jnp.sum(adj, axis=1, keepdims=True), 1.0)
        invdeg = 1.0 / deg
        m = jnp.dot(adj.astype(jnp.bfloat16), g_ref[k],
                    preferred_element_type=jnp.float32)
        h = jnp.maximum(m * invdeg + b1_ref[...], 0.0)
        c = jax.lax.dot_general(invdeg, adj, (((0,), (0,)), ((), ())),
                                preferred_element_type=jnp.float32)
        y = jnp.dot(c, h, preferred_element_type=jnp.float32) * (1.0 / s)
        y = jnp.dot(y, w2_ref[...],
                    preferred_element_type=jnp.float32) + b2_ref[...]
        nrm = jnp.sqrt(jnp.sum(y * y))
        out_ref[k] = y / jnp.maximum(nrm, 1e-5)


@jax.jit
def kernel(adj, n_feat, W1, b1, W2, b2):
    B, S, _ = adj.shape
    FT = n_feat.shape[-1]
    H = W1.shape[-1]
    O = W2.shape[-1]
    b1r = b1.reshape(1, H)
    b2r = b2.reshape(1, O)
    vmem = pl.BlockSpec(memory_space=pltpu.MemorySpace.VMEM)
    return pl.pallas_call(
        _gcn_body,
        in_specs=[
            pl.BlockSpec(memory_space=pltpu.MemorySpace.HBM),
            vmem, vmem, vmem, vmem, vmem,
        ],
        out_specs=vmem,
        out_shape=jax.ShapeDtypeStruct((B, 1, O), jnp.float32),
        scratch_shapes=[
            pltpu.VMEM((2, S, S), jnp.float32),
            pltpu.VMEM((B, S, H), jnp.bfloat16),
            pltpu.SemaphoreType.DMA((2,)),
        ],
        compiler_params=pltpu.CompilerParams(
            vmem_limit_bytes=120 * 1024 * 1024),
    )(adj, n_feat, W1, b1r, W2, b2r).reshape(B, O)


# speedup vs baseline: 1.0178x; 1.0178x over previous
import jax
import jax.numpy as jnp
from jax.experimental import pallas as pl
from jax.experimental.pallas import tpu as pltpu


def _one_batch(adj, g, w2, b1, b2):
    s = adj.shape[-1]
    deg = jnp.maximum(jnp.sum(adj, axis=1, keepdims=True), 1.0)
    invdeg = 1.0 / deg
    m = jnp.dot(adj, g, preferred_element_type=jnp.float32)
    h = jnp.maximum(m * invdeg + b1, 0.0)
    c = jax.lax.dot_general(invdeg, adj, (((0,), (0,)), ((), ())),
                            preferred_element_type=jnp.float32)
    y = jnp.dot(c, h, preferred_element_type=jnp.float32) * (1.0 / s)
    y = jnp.dot(y, w2, preferred_element_type=jnp.float32) + b2
    nrm = jnp.sqrt(jnp.sum(y * y))
    return y / jnp.maximum(nrm, 1e-5)


def _gcn_body(adj0_ref, adj1_ref, feat_ref, w1_ref, b1_ref, w2_ref, b2_ref,
              out_ref):
    feat = feat_ref[0]
    g = jax.lax.dot_general(feat, w1_ref[...], (((2,), (0,)), ((), ())),
                            preferred_element_type=jnp.float32)
    b1 = b1_ref[...]
    b2 = b2_ref[...]
    w2 = w2_ref[...]
    out_ref[0] = _one_batch(adj0_ref[0, 0], g[0], w2, b1, b2)
    out_ref[1] = _one_batch(adj1_ref[0, 0], g[1], w2, b1, b2)


@jax.jit
def kernel(adj, n_feat, W1, b1, W2, b2):
    B, S, _ = adj.shape
    FT = n_feat.shape[-1]
    H = W1.shape[-1]
    O = W2.shape[-1]
    b1r = b1.reshape(1, H)
    b2r = b2.reshape(1, O)
    adj2 = adj.reshape(B // 2, 2, S, S)
    feat2 = n_feat.reshape(B // 2, 2, S, FT)
    return pl.pallas_call(
        _gcn_body,
        grid=(B // 2,),
        in_specs=[
            pl.BlockSpec((1, 1, S, S), lambda b: (b, 0, 0, 0)),
            pl.BlockSpec((1, 1, S, S), lambda b: (b, 1, 0, 0)),
            pl.BlockSpec((1, 2, S, FT), lambda b: (b, 0, 0, 0)),
            pl.BlockSpec((FT, H), lambda b: (0, 0)),
            pl.BlockSpec((1, H), lambda b: (0, 0)),
            pl.BlockSpec((H, O), lambda b: (0, 0)),
            pl.BlockSpec((1, O), lambda b: (0, 0)),
        ],
        out_specs=pl.BlockSpec((2, 1, O), lambda b: (b, 0, 0)),
        out_shape=jax.ShapeDtypeStruct((B, 1, O), jnp.float32),
        compiler_params=pltpu.CompilerParams(
            dimension_semantics=("parallel",),
            vmem_limit_bytes=120 * 1024 * 1024),
    )(adj2, adj2, feat2, W1, b1r, W2, b2r).reshape(B, O)


# speedup vs baseline: 1.1579x; 1.1376x over previous
import jax
import jax.numpy as jnp
from jax.experimental import pallas as pl
from jax.experimental.pallas import tpu as pltpu

_BPB = 4


def _gcn_body(adj_ref, feat_ref, w1_ref, b1_ref, w2_ref, b2_ref, out_ref):
    adj = adj_ref[...]
    feat = feat_ref[...]
    s = adj.shape[-1]
    deg = jnp.maximum(jnp.sum(adj, axis=2, keepdims=True), 1.0)
    invdeg = 1.0 / deg
    g = jax.lax.dot_general(feat, w1_ref[...], (((2,), (0,)), ((), ())),
                            preferred_element_type=jnp.float32)
    m = jax.lax.dot_general(adj, g, (((2,), (1,)), ((0,), (0,))),
                            preferred_element_type=jnp.float32)
    h = jnp.maximum(m * invdeg + b1_ref[...], 0.0)
    c = jnp.sum(adj * invdeg, axis=1, keepdims=True)
    y = jax.lax.dot_general(c, h, (((2,), (1,)), ((0,), (0,))),
                            preferred_element_type=jnp.float32) * (1.0 / s)
    y = jax.lax.dot_general(y, w2_ref[...], (((2,), (0,)), ((), ())),
                            preferred_element_type=jnp.float32) + b2_ref[...]
    nrm = jnp.sqrt(jnp.sum(y * y, axis=-1, keepdims=True))
    out_ref[...] = y / jnp.maximum(nrm, 1e-5)


@jax.jit
def kernel(adj, n_feat, W1, b1, W2, b2):
    B, S, _ = adj.shape
    FT = n_feat.shape[-1]
    H = W1.shape[-1]
    O = W2.shape[-1]
    b1r = b1.reshape(1, H)
    b2r = b2.reshape(1, O)
    return pl.pallas_call(
        _gcn_body,
        grid=(B // _BPB,),
        in_specs=[
            pl.BlockSpec((_BPB, S, S), lambda b: (b, 0, 0)),
            pl.BlockSpec((_BPB, S, FT), lambda b: (b, 0, 0)),
            pl.BlockSpec((FT, H), lambda b: (0, 0)),
            pl.BlockSpec((1, H), lambda b: (0, 0)),
            pl.BlockSpec((H, O), lambda b: (0, 0)),
            pl.BlockSpec((1, O), lambda b: (0, 0)),
        ],
        out_specs=pl.BlockSpec((_BPB, 1, O), lambda b: (b, 0, 0)),
        out_shape=jax.ShapeDtypeStruct((B, 1, O), jnp.float32),
        compiler_params=pltpu.CompilerParams(
            dimension_semantics=("parallel",),
            vmem_limit_bytes=120 * 1024 * 1024),
    )(adj, n_feat, W1, b1r, W2, b2r).reshape(B, O)


# speedup vs baseline: 1.1744x; 1.0142x over previous
import jax
import jax.numpy as jnp
from jax.experimental import pallas as pl
from jax.experimental.pallas import tpu as pltpu

_BPB = 4


def _gcn_body(adj_ref, feat_ref, w1_ref, b1_ref, w2_ref, b2_ref, out_ref):
    adj = adj_ref[...]
    feat = feat_ref[...]
    s = adj.shape[-1]
    adjb = adj.astype(jnp.bfloat16)
    deg = jnp.maximum(jnp.sum(adj, axis=2, keepdims=True), 1.0)
    invdeg = 1.0 / deg
    g = jax.lax.dot_general(feat, w1_ref[...], (((2,), (0,)), ((), ())),
                            preferred_element_type=jnp.float32)
    m = jax.lax.dot_general(adjb, g.astype(jnp.bfloat16),
                            (((2,), (1,)), ((0,), (0,))),
                            preferred_element_type=jnp.float32)
    h = jnp.maximum(m * invdeg + b1_ref[...], 0.0)
    c = jax.lax.dot_general(invdeg.astype(jnp.bfloat16), adjb,
                            (((1,), (1,)), ((0,), (0,))),
                            preferred_element_type=jnp.float32)
    y = jax.lax.dot_general(c, h, (((2,), (1,)), ((0,), (0,))),
                            preferred_element_type=jnp.float32) * (1.0 / s)
    y = jax.lax.dot_general(y, w2_ref[...], (((2,), (0,)), ((), ())),
                            preferred_element_type=jnp.float32) + b2_ref[...]
    nrm = jnp.sqrt(jnp.sum(y * y, axis=-1, keepdims=True))
    out_ref[...] = y / jnp.maximum(nrm, 1e-5)


@jax.jit
def kernel(adj, n_feat, W1, b1, W2, b2):
    B, S, _ = adj.shape
    FT = n_feat.shape[-1]
    H = W1.shape[-1]
    O = W2.shape[-1]
    b1r = b1.reshape(1, H)
    b2r = b2.reshape(1, O)
    return pl.pallas_call(
        _gcn_body,
        grid=(B // _BPB,),
        in_specs=[
            pl.BlockSpec((_BPB, S, S), lambda b: (b, 0, 0)),
            pl.BlockSpec((_BPB, S, FT), lambda b: (b, 0, 0)),
            pl.BlockSpec((FT, H), lambda b: (0, 0)),
            pl.BlockSpec((1, H), lambda b: (0, 0)),
            pl.BlockSpec((H, O), lambda b: (0, 0)),
            pl.BlockSpec((1, O), lambda b: (0, 0)),
        ],
        out_specs=pl.BlockSpec((_BPB, 1, O), lambda b: (b, 0, 0)),
        out_shape=jax.ShapeDtypeStruct((B, 1, O), jnp.float32),
        compiler_params=pltpu.CompilerParams(
            dimension_semantics=("parallel",),
            vmem_limit_bytes=120 * 1024 * 1024),
    )(adj, n_feat, W1, b1r, W2, b2r).reshape(B, O)


# speedup vs baseline: 1.1847x; 1.0088x over previous
import jax
import jax.numpy as jnp
from jax.experimental import pallas as pl
from jax.experimental.pallas import tpu as pltpu

_BPB = 4


def _gcn_body(adj_ref, feat_ref, w1_ref, b1_ref, w2_ref, b2_ref, out_ref):
    adj = adj_ref[...]
    feat = feat_ref[...]
    s = adj.shape[-1]
    g = jax.lax.dot_general(feat, w1_ref[...], (((2,), (0,)), ((), ())),
                            preferred_element_type=jnp.float32)
    ones_col = jnp.ones(g.shape[:-1] + (1,), dtype=jnp.float32)
    g_aug = jnp.concatenate([g, ones_col], axis=-1)
    m_aug = jax.lax.dot_general(adj, g_aug, (((2,), (1,)), ((0,), (0,))),
                                preferred_element_type=jnp.float32)
    h_dim = g.shape[-1]
    deg = jnp.maximum(m_aug[:, :, h_dim:h_dim + 1], 1.0)
    invdeg = 1.0 / deg
    m = m_aug[:, :, :h_dim]
    h = jnp.maximum(m * invdeg + b1_ref[...], 0.0)
    c = jax.lax.dot_general(invdeg, adj, (((1,), (1,)), ((0,), (0,))),
                            preferred_element_type=jnp.float32)
    y = jax.lax.dot_general(c, h, (((2,), (1,)), ((0,), (0,))),
                            preferred_element_type=jnp.float32) * (1.0 / s)
    y = jax.lax.dot_general(y, w2_ref[...], (((2,), (0,)), ((), ())),
                            preferred_element_type=jnp.float32) + b2_ref[...]
    nrm = jnp.sqrt(jnp.sum(y * y, axis=-1, keepdims=True))
    out_ref[...] = y / jnp.maximum(nrm, 1e-5)


@jax.jit
def kernel(adj, n_feat, W1, b1, W2, b2):
    B, S, _ = adj.shape
    FT = n_feat.shape[-1]
    H = W1.shape[-1]
    O = W2.shape[-1]
    b1r = b1.reshape(1, H)
    b2r = b2.reshape(1, O)
    return pl.pallas_call(
        _gcn_body,
        grid=(B // _BPB,),
        in_specs=[
            pl.BlockSpec((_BPB, S, S), lambda b: (b, 0, 0)),
            pl.BlockSpec((_BPB, S, FT), lambda b: (b, 0, 0)),
            pl.BlockSpec((FT, H), lambda b: (0, 0)),
            pl.BlockSpec((1, H), lambda b: (0, 0)),
            pl.BlockSpec((H, O), lambda b: (0, 0)),
            pl.BlockSpec((1, O), lambda b: (0, 0)),
        ],
        out_specs=pl.BlockSpec((_BPB, 1, O), lambda b: (b, 0, 0)),
        out_shape=jax.ShapeDtypeStruct((B, 1, O), jnp.float32),
        compiler_params=pltpu.CompilerParams(
            dimension_semantics=("parallel",),
            vmem_limit_bytes=120 * 1024 * 1024),
    )(adj, n_feat, W1, b1r, W2, b2r).reshape(B, O)
